# Initial kernel scaffold; baseline (speedup 1.0000x reference)
import jax
import jax.numpy as jnp
from jax.experimental import pallas as pl


def kernel(x, edge_index, batch, W1, b1, W2, b2, W3, b3):
    raise NotImplementedError("write your pallas kernel here")


# speedup vs baseline: 11.9493x; 11.9493x over previous
import functools

import jax
import jax.numpy as jnp
from jax import lax
from jax.experimental import pallas as pl
from jax.experimental.pallas import tpu as pltpu
from jax.experimental.pallas import tpu_sc as plsc

NC = 2
NS = 16
CK = 128


def _sc_mesh():
    return plsc.VectorSubcoreMesh(
        core_axis_name="c", subcore_axis_name="s",
        num_cores=NC, num_subcores=NS)


_SC_PARAMS = pltpu.CompilerParams(use_tc_tiling_on_sc=False)


@functools.lru_cache(maxsize=None)
def _deg_pass(N, E):
    TE = E // (NC * NS)
    nfull, tail = TE // CK, TE % CK
    RT = N // NS

    @functools.partial(
        pl.kernel,
        out_type=jax.ShapeDtypeStruct((NC, N, 1), jnp.float32),
        mesh=_sc_mesh(),
        compiler_params=_SC_PARAMS,
        scratch_types=[
            pltpu.VMEM((CK,), jnp.int32),
            pltpu.VMEM((tail,), jnp.int32),
            pltpu.VMEM((CK, 1), jnp.float32),
            pltpu.VMEM_SHARED((N, 1), jnp.float32),
        ],
    )
    def kern(col_hbm, ones_hbm, out, cidx, cidx_t, ones_v, acc):
        c = lax.axis_index("c")
        s = lax.axis_index("s")
        r0 = s * RT
        pltpu.sync_copy(ones_hbm.at[pl.ds(r0, RT)], acc.at[pl.ds(r0, RT)])
        pltpu.sync_copy(ones_hbm.at[pl.ds(0, CK)], ones_v)
        plsc.subcore_barrier()
        base = (s * NC + c) * TE

        def body(i, carry):
            pltpu.sync_copy(col_hbm.at[pl.ds(base + i * CK, CK)], cidx)
            pltpu.sync_copy(ones_v, acc.at[cidx], add=True)
            return carry

        lax.fori_loop(0, nfull, body, 0)
        if tail:
            pltpu.sync_copy(col_hbm.at[pl.ds(base + nfull * CK, tail)], cidx_t)
            pltpu.sync_copy(ones_v.at[pl.ds(0, tail)], acc.at[cidx_t], add=True)
        plsc.subcore_barrier()
        pltpu.sync_copy(acc.at[pl.ds(r0, RT)], out.at[c, pl.ds(r0, RT)])

    return kern


@functools.lru_cache(maxsize=None)
def _edge_pass(N, E, D):
    TE = E // NS
    nfull, tail = TE // CK, TE % CK
    RT = N // NS

    @functools.partial(
        pl.kernel,
        out_type=(jax.ShapeDtypeStruct((N, D), jnp.float32),
                  jax.ShapeDtypeStruct((N, D), jnp.float32)),
        mesh=_sc_mesh(),
        compiler_params=_SC_PARAMS,
        scratch_types=[
            pltpu.VMEM((CK,), jnp.int32),
            pltpu.VMEM((CK,), jnp.int32),
            pltpu.VMEM((CK, D), jnp.float32),
            pltpu.VMEM((tail,), jnp.int32),
            pltpu.VMEM((tail,), jnp.int32),
            pltpu.VMEM((tail, D), jnp.float32),
            pltpu.VMEM_SHARED((N, D), jnp.float32),
            pltpu.SemaphoreType.DMA,
        ],
    )
    def kern(g_lo, g_hi, row_hbm, col_hbm, out_lo, out_hi,
             ridx, cidx, msg, ridx_t, cidx_t, msg_t, acc, sem):
        c = lax.axis_index("c")
        s = lax.axis_index("s")

        def run(gref, oref):
            r0 = s * RT
            pltpu.sync_copy(gref.at[pl.ds(r0, RT)], acc.at[pl.ds(r0, RT)])
            plsc.subcore_barrier()
            ebase = s * TE

            def chunk(off, rbuf, cbuf, mbuf, n):
                pltpu.sync_copy(row_hbm.at[pl.ds(off, n)], rbuf)
                pltpu.sync_copy(col_hbm.at[pl.ds(off, n)], cbuf)
                pltpu.async_copy(gref.at[rbuf], mbuf, sem).wait()
                pltpu.sync_copy(mbuf, acc.at[cbuf], add=True)

            def body(i, carry):
                chunk(ebase + i * CK, ridx, cidx, msg, CK)
                return carry

            lax.fori_loop(0, nfull, body, 0)
            if tail:
                chunk(ebase + nfull * CK, ridx_t, cidx_t, msg_t, tail)
            plsc.subcore_barrier()
            pltpu.sync_copy(acc.at[pl.ds(r0, RT)], oref.at[pl.ds(r0, RT)])

        @pl.when(c == 0)
        def _():
            run(g_lo, out_lo)

        @pl.when(c == 1)
        def _():
            run(g_hi, out_hi)

    return kern


BLK = 5000


@functools.lru_cache(maxsize=None)
def _prep1(N, F):
    H = F // 2
    grid = (N // BLK,)

    def body(x_ref, w_ref, p0_ref, p1_ref, lo_ref, hi_ref, dinv_ref):
        deg = p0_ref[...] + p1_ref[...] - 1.0
        dinv = lax.rsqrt(deg)
        h = lax.dot_general(x_ref[...], w_ref[...],
                            (((1,), (0,)), ((), ())),
                            preferred_element_type=jnp.float32)
        g = h * dinv
        lo_ref[...] = g[:, :H]
        hi_ref[...] = g[:, H:]
        dinv_ref[...] = dinv

    return pl.pallas_call(
        body,
        grid=grid,
        in_specs=[
            pl.BlockSpec((BLK, 8), lambda i: (i, 0)),
            pl.BlockSpec((8, F), lambda i: (0, 0)),
            pl.BlockSpec((BLK, 1), lambda i: (i, 0)),
            pl.BlockSpec((BLK, 1), lambda i: (i, 0)),
        ],
        out_specs=[
            pl.BlockSpec((BLK, H), lambda i: (i, 0)),
            pl.BlockSpec((BLK, H), lambda i: (i, 0)),
            pl.BlockSpec((BLK, 1), lambda i: (i, 0)),
        ],
        out_shape=[
            jax.ShapeDtypeStruct((N, H), jnp.float32),
            jax.ShapeDtypeStruct((N, H), jnp.float32),
            jax.ShapeDtypeStruct((N, 1), jnp.float32),
        ],
    )


@functools.lru_cache(maxsize=None)
def _mid(N, F1, F2):
    H1, H2 = F1 // 2, F2 // 2
    grid = (N // BLK,)

    def body(lo_ref, hi_ref, dinv_ref, b1_ref, w2_ref, olo_ref, ohi_ref):
        accv = jnp.concatenate([lo_ref[...], hi_ref[...]], axis=1)
        dinv = dinv_ref[...]
        out1 = jnp.maximum(accv * dinv + b1_ref[...], 0.0)
        h2 = lax.dot_general(out1, w2_ref[...],
                             (((1,), (0,)), ((), ())),
                             preferred_element_type=jnp.float32)
        g2 = h2 * dinv
        olo_ref[...] = g2[:, :H2]
        ohi_ref[...] = g2[:, H2:]

    return pl.pallas_call(
        body,
        grid=grid,
        in_specs=[
            pl.BlockSpec((BLK, H1), lambda i: (i, 0)),
            pl.BlockSpec((BLK, H1), lambda i: (i, 0)),
            pl.BlockSpec((BLK, 1), lambda i: (i, 0)),
            pl.BlockSpec((1, F1), lambda i: (0, 0)),
            pl.BlockSpec((F1, F2), lambda i: (0, 0)),
        ],
        out_specs=[
            pl.BlockSpec((BLK, H2), lambda i: (i, 0)),
            pl.BlockSpec((BLK, H2), lambda i: (i, 0)),
        ],
        out_shape=[
            jax.ShapeDtypeStruct((N, H2), jnp.float32),
            jax.ShapeDtypeStruct((N, H2), jnp.float32),
        ],
    )


@functools.lru_cache(maxsize=None)
def _tail(N, F2, G):
    H2 = F2 // 2
    grid = (N // BLK,)
    ng = N // BLK

    def body(lo_ref, hi_ref, dinv_ref, b2_ref, bat_ref, w3_ref, b3_ref,
             out_ref, acc_ref):
        pid = pl.program_id(0)

        @pl.when(pid == 0)
        def _():
            acc_ref[...] = jnp.zeros_like(acc_ref)

        accv = jnp.concatenate([lo_ref[...], hi_ref[...]], axis=1)
        h = jnp.maximum(accv * dinv_ref[...] + b2_ref[...], 0.0)
        h1 = jnp.concatenate([h, jnp.ones((BLK, 1), jnp.float32)], axis=1)
        seg = lax.broadcasted_iota(jnp.int32, (1, G), 1)
        onehot = (bat_ref[...] == seg).astype(jnp.float32)
        acc_ref[...] += lax.dot_general(onehot, h1,
                                        (((0,), (0,)), ((), ())),
                                        preferred_element_type=jnp.float32)

        @pl.when(pid == ng - 1)
        def _():
            a = acc_ref[...]
            sums = a[:, :F2]
            cnt = a[:, F2:F2 + 1]
            pooled = sums / jnp.maximum(cnt, 1.0)
            z = lax.dot_general(pooled, w3_ref[...],
                                (((1,), (0,)), ((), ())),
                                preferred_element_type=jnp.float32) + b3_ref[...]
            out_ref[...] = 1.0 / (1.0 + jnp.exp(-z))

    return pl.pallas_call(
        body,
        grid=grid,
        in_specs=[
            pl.BlockSpec((BLK, H2), lambda i: (i, 0)),
            pl.BlockSpec((BLK, H2), lambda i: (i, 0)),
            pl.BlockSpec((BLK, 1), lambda i: (i, 0)),
            pl.BlockSpec((1, F2), lambda i: (0, 0)),
            pl.BlockSpec((BLK, 1), lambda i: (i, 0)),
            pl.BlockSpec((F2, 1), lambda i: (0, 0)),
            pl.BlockSpec((1, 1), lambda i: (0, 0)),
        ],
        out_specs=pl.BlockSpec((G, 1), lambda i: (0, 0)),
        out_shape=jax.ShapeDtypeStruct((G, 1), jnp.float32),
        scratch_shapes=[pltpu.VMEM((G, F2 + 1), jnp.float32)],
    )


def kernel(x, edge_index, batch, W1, b1, W2, b2, W3, b3):
    N = x.shape[0]
    E = edge_index.shape[1]
    G = 64
    row = edge_index[0]
    col = edge_index[1]
    Np = ((N + 127) // 128) * 128
    pad = ((0, Np - N), (0, 0))
    ones = jnp.ones((Np, 1), jnp.float32)

    degp = _deg_pass(Np, E)(col, ones)
    p0 = degp[0, :N]
    p1 = degp[1, :N]
    g1lo, g1hi, dinv = _prep1(N, 16)(x, W1, p0, p1)
    s1lo, s1hi = _edge_pass(Np, E, 8)(jnp.pad(g1lo, pad), jnp.pad(g1hi, pad),
                                      row, col)
    s1lo, s1hi = s1lo[:N], s1hi[:N]
    g2lo, g2hi = _mid(N, 16, 24)(s1lo, s1hi, dinv, b1.reshape(1, 16), W2)
    s2lo, s2hi = _edge_pass(Np, E, 12)(jnp.pad(g2lo, pad), jnp.pad(g2hi, pad),
                                       row, col)
    s2lo, s2hi = s2lo[:N], s2hi[:N]
    out = _tail(N, 24, G)(s2lo, s2hi, dinv, b2.reshape(1, 24),
                          batch.reshape(N, 1), W3, b3.reshape(1, 1))
    return out


# speedup vs baseline: 33.7686x; 2.8260x over previous
import functools

import jax
import jax.numpy as jnp
from jax import lax
from jax.experimental import pallas as pl
from jax.experimental.pallas import tpu as pltpu
from jax.experimental.pallas import tpu_sc as plsc

NC = 2
NS = 16
CK = 128
SB = 8


def _sc_mesh():
    return plsc.VectorSubcoreMesh(
        core_axis_name="c", subcore_axis_name="s",
        num_cores=NC, num_subcores=NS)


_SC_PARAMS = pltpu.CompilerParams(use_tc_tiling_on_sc=False)


@functools.lru_cache(maxsize=None)
def _deg_pass(N, E):
    EC = E // CK
    CPT = EC // (NC * NS)
    nss, rem = CPT // SB, CPT % SB
    XT = EC - CPT * NC * NS
    RT = N // NS

    @functools.partial(
        pl.kernel,
        out_type=jax.ShapeDtypeStruct((NC, N, 1), jnp.float32),
        mesh=_sc_mesh(),
        compiler_params=_SC_PARAMS,
        scratch_types=[
            pltpu.VMEM((SB, CK), jnp.int32),
            pltpu.VMEM((CK, 1), jnp.float32),
            pltpu.VMEM_SHARED((N, 1), jnp.float32),
            pltpu.SemaphoreType.DMA,
        ],
    )
    def kern(col2_hbm, ones_hbm, out, cbuf, ones_v, acc, ssem):
        c = lax.axis_index("c")
        s = lax.axis_index("s")
        r0 = s * RT
        pltpu.sync_copy(ones_hbm.at[pl.ds(r0, RT)], acc.at[pl.ds(r0, RT)])
        pltpu.sync_copy(ones_hbm.at[pl.ds(0, CK)], ones_v)
        plsc.subcore_barrier()
        wid = s * NC + c

        def superstep(cb, nb):
            pltpu.sync_copy(col2_hbm.at[pl.ds(cb, nb)], cbuf.at[pl.ds(0, nb)])
            sd = [pltpu.async_copy(ones_v, acc.at[cbuf.at[j]], ssem, add=True)
                  for j in range(nb)]
            for d in sd:
                d.wait()

        cb0 = wid * CPT

        def body(i, carry):
            superstep(cb0 + i * SB, SB)
            return carry

        lax.fori_loop(0, nss, body, 0)
        if rem:
            superstep(cb0 + nss * SB, rem)
        if XT:
            @pl.when(wid < XT)
            def _():
                superstep(EC - XT + wid, 1)
        plsc.subcore_barrier()
        pltpu.sync_copy(acc.at[pl.ds(r0, RT)], out.at[c, pl.ds(r0, RT)])

    return kern


@functools.lru_cache(maxsize=None)
def _edge_pass(N, E, D):
    EC = E // CK
    CPT = EC // NS
    nss, rem = CPT // SB, CPT % SB
    XT = EC - CPT * NS
    RT = N // NS

    @functools.partial(
        pl.kernel,
        out_type=(jax.ShapeDtypeStruct((N, D), jnp.float32),
                  jax.ShapeDtypeStruct((N, D), jnp.float32)),
        mesh=_sc_mesh(),
        compiler_params=_SC_PARAMS,
        scratch_types=[
            pltpu.VMEM((SB, CK), jnp.int32),
            pltpu.VMEM((SB, CK), jnp.int32),
            pltpu.VMEM((SB, CK, D), jnp.float32),
            pltpu.VMEM_SHARED((N, D), jnp.float32),
            pltpu.SemaphoreType.DMA,
            pltpu.SemaphoreType.DMA,
        ],
    )
    def kern(g_lo, g_hi, row2_hbm, col2_hbm, out_lo, out_hi,
             rbuf, cbuf, msg, acc, gsem, ssem):
        c = lax.axis_index("c")
        s = lax.axis_index("s")

        def run(gref, oref):
            r0 = s * RT
            pltpu.sync_copy(gref.at[pl.ds(r0, RT)], acc.at[pl.ds(r0, RT)])
            plsc.subcore_barrier()

            def superstep(cb, nb):
                pltpu.sync_copy(row2_hbm.at[pl.ds(cb, nb)],
                                rbuf.at[pl.ds(0, nb)])
                pltpu.sync_copy(col2_hbm.at[pl.ds(cb, nb)],
                                cbuf.at[pl.ds(0, nb)])
                gd = [pltpu.async_copy(gref.at[rbuf.at[j]], msg.at[j], gsem)
                      for j in range(nb)]
                for d in gd:
                    d.wait()
                sd = [pltpu.async_copy(msg.at[j], acc.at[cbuf.at[j]], ssem,
                                       add=True)
                      for j in range(nb)]
                for d in sd:
                    d.wait()

            cb0 = s * CPT

            def body(i, carry):
                superstep(cb0 + i * SB, SB)
                return carry

            lax.fori_loop(0, nss, body, 0)
            if rem:
                superstep(cb0 + nss * SB, rem)
            if XT:
                @pl.when(s < XT)
                def _():
                    superstep(EC - XT + s, 1)
            plsc.subcore_barrier()
            pltpu.sync_copy(acc.at[pl.ds(r0, RT)], oref.at[pl.ds(r0, RT)])

        @pl.when(c == 0)
        def _():
            run(g_lo, out_lo)

        @pl.when(c == 1)
        def _():
            run(g_hi, out_hi)

    return kern


BLK = 5000


@functools.lru_cache(maxsize=None)
def _prep1(N, F):
    H = F // 2
    grid = (N // BLK,)

    def body(x_ref, w_ref, p0_ref, p1_ref, lo_ref, hi_ref, dinv_ref):
        deg = p0_ref[...] + p1_ref[...] - 1.0
        dinv = lax.rsqrt(deg)
        h = lax.dot_general(x_ref[...], w_ref[...],
                            (((1,), (0,)), ((), ())),
                            preferred_element_type=jnp.float32)
        g = h * dinv
        lo_ref[...] = g[:, :H]
        hi_ref[...] = g[:, H:]
        dinv_ref[...] = dinv

    return pl.pallas_call(
        body,
        grid=grid,
        in_specs=[
            pl.BlockSpec((BLK, 8), lambda i: (i, 0)),
            pl.BlockSpec((8, F), lambda i: (0, 0)),
            pl.BlockSpec((BLK, 1), lambda i: (i, 0)),
            pl.BlockSpec((BLK, 1), lambda i: (i, 0)),
        ],
        out_specs=[
            pl.BlockSpec((BLK, H), lambda i: (i, 0)),
            pl.BlockSpec((BLK, H), lambda i: (i, 0)),
            pl.BlockSpec((BLK, 1), lambda i: (i, 0)),
        ],
        out_shape=[
            jax.ShapeDtypeStruct((N, H), jnp.float32),
            jax.ShapeDtypeStruct((N, H), jnp.float32),
            jax.ShapeDtypeStruct((N, 1), jnp.float32),
        ],
    )


@functools.lru_cache(maxsize=None)
def _mid(N, F1, F2):
    H1, H2 = F1 // 2, F2 // 2
    grid = (N // BLK,)

    def body(lo_ref, hi_ref, dinv_ref, b1_ref, w2_ref, olo_ref, ohi_ref):
        accv = jnp.concatenate([lo_ref[...], hi_ref[...]], axis=1)
        dinv = dinv_ref[...]
        out1 = jnp.maximum(accv * dinv + b1_ref[...], 0.0)
        h2 = lax.dot_general(out1, w2_ref[...],
                             (((1,), (0,)), ((), ())),
                             preferred_element_type=jnp.float32)
        g2 = h2 * dinv
        olo_ref[...] = g2[:, :H2]
        ohi_ref[...] = g2[:, H2:]

    return pl.pallas_call(
        body,
        grid=grid,
        in_specs=[
            pl.BlockSpec((BLK, H1), lambda i: (i, 0)),
            pl.BlockSpec((BLK, H1), lambda i: (i, 0)),
            pl.BlockSpec((BLK, 1), lambda i: (i, 0)),
            pl.BlockSpec((1, F1), lambda i: (0, 0)),
            pl.BlockSpec((F1, F2), lambda i: (0, 0)),
        ],
        out_specs=[
            pl.BlockSpec((BLK, H2), lambda i: (i, 0)),
            pl.BlockSpec((BLK, H2), lambda i: (i, 0)),
        ],
        out_shape=[
            jax.ShapeDtypeStruct((N, H2), jnp.float32),
            jax.ShapeDtypeStruct((N, H2), jnp.float32),
        ],
    )


@functools.lru_cache(maxsize=None)
def _tail(N, F2, G):
    H2 = F2 // 2
    grid = (N // BLK,)
    ng = N // BLK

    def body(lo_ref, hi_ref, dinv_ref, b2_ref, bat_ref, w3_ref, b3_ref,
             out_ref, acc_ref):
        pid = pl.program_id(0)

        @pl.when(pid == 0)
        def _():
            acc_ref[...] = jnp.zeros_like(acc_ref)

        accv = jnp.concatenate([lo_ref[...], hi_ref[...]], axis=1)
        h = jnp.maximum(accv * dinv_ref[...] + b2_ref[...], 0.0)
        h1 = jnp.concatenate([h, jnp.ones((BLK, 1), jnp.float32)], axis=1)
        seg = lax.broadcasted_iota(jnp.int32, (1, G), 1)
        onehot = (bat_ref[...] == seg).astype(jnp.float32)
        acc_ref[...] += lax.dot_general(onehot, h1,
                                        (((0,), (0,)), ((), ())),
                                        preferred_element_type=jnp.float32)

        @pl.when(pid == ng - 1)
        def _():
            a = acc_ref[...]
            sums = a[:, :F2]
            cnt = a[:, F2:F2 + 1]
            pooled = sums / jnp.maximum(cnt, 1.0)
            z = lax.dot_general(pooled, w3_ref[...],
                                (((1,), (0,)), ((), ())),
                                preferred_element_type=jnp.float32) + b3_ref[...]
            out_ref[...] = 1.0 / (1.0 + jnp.exp(-z))

    return pl.pallas_call(
        body,
        grid=grid,
        in_specs=[
            pl.BlockSpec((BLK, H2), lambda i: (i, 0)),
            pl.BlockSpec((BLK, H2), lambda i: (i, 0)),
            pl.BlockSpec((BLK, 1), lambda i: (i, 0)),
            pl.BlockSpec((1, F2), lambda i: (0, 0)),
            pl.BlockSpec((BLK, 1), lambda i: (i, 0)),
            pl.BlockSpec((F2, 1), lambda i: (0, 0)),
            pl.BlockSpec((1, 1), lambda i: (0, 0)),
        ],
        out_specs=pl.BlockSpec((G, 1), lambda i: (0, 0)),
        out_shape=jax.ShapeDtypeStruct((G, 1), jnp.float32),
        scratch_shapes=[pltpu.VMEM((G, F2 + 1), jnp.float32)],
    )


def kernel(x, edge_index, batch, W1, b1, W2, b2, W3, b3):
    N = x.shape[0]
    E = edge_index.shape[1]
    G = 64
    row2 = edge_index[0].reshape(E // CK, CK)
    col2 = edge_index[1].reshape(E // CK, CK)
    Np = ((N + 127) // 128) * 128
    pad = ((0, Np - N), (0, 0))
    ones = jnp.ones((Np, 1), jnp.float32)

    degp = _deg_pass(Np, E)(col2, ones)
    p0 = degp[0, :N]
    p1 = degp[1, :N]
    g1lo, g1hi, dinv = _prep1(N, 16)(x, W1, p0, p1)
    s1lo, s1hi = _edge_pass(Np, E, 8)(jnp.pad(g1lo, pad), jnp.pad(g1hi, pad),
                                      row2, col2)
    s1lo, s1hi = s1lo[:N], s1hi[:N]
    g2lo, g2hi = _mid(N, 16, 24)(s1lo, s1hi, dinv, b1.reshape(1, 16), W2)
    s2lo, s2hi = _edge_pass(Np, E, 12)(jnp.pad(g2lo, pad), jnp.pad(g2hi, pad),
                                       row2, col2)
    s2lo, s2hi = s2lo[:N], s2hi[:N]
    out = _tail(N, 24, G)(s2lo, s2hi, dinv, b2.reshape(1, 24),
                          batch.reshape(N, 1), W3, b3.reshape(1, 1))
    return out


# speedup vs baseline: 36.6542x; 1.0854x over previous
import functools

import jax
import jax.numpy as jnp
from jax import lax
from jax.experimental import pallas as pl
from jax.experimental.pallas import tpu as pltpu
from jax.experimental.pallas import tpu_sc as plsc

NC = 2
NS = 16
CK = 128
SB = 8


def _sc_mesh():
    return plsc.VectorSubcoreMesh(
        core_axis_name="c", subcore_axis_name="s",
        num_cores=NC, num_subcores=NS)


_SC_PARAMS = pltpu.CompilerParams(use_tc_tiling_on_sc=False)


@functools.lru_cache(maxsize=None)
def _deg_pass(N, E):
    EC = E // CK
    CPT = EC // (NC * NS)
    nss, rem = CPT // SB, CPT % SB
    XT = EC - CPT * NC * NS
    RT = N // NS

    @functools.partial(
        pl.kernel,
        out_type=jax.ShapeDtypeStruct((NC, N, 1), jnp.float32),
        mesh=_sc_mesh(),
        compiler_params=_SC_PARAMS,
        scratch_types=[
            pltpu.VMEM((SB, CK), jnp.int32),
            pltpu.VMEM((CK, 1), jnp.float32),
            pltpu.VMEM_SHARED((N, 1), jnp.float32),
            pltpu.SemaphoreType.DMA,
        ],
    )
    def kern(col2_hbm, ones_hbm, out, cbuf, ones_v, acc, ssem):
        c = lax.axis_index("c")
        s = lax.axis_index("s")
        r0 = s * RT
        pltpu.sync_copy(ones_hbm.at[pl.ds(r0, RT)], acc.at[pl.ds(r0, RT)])
        pltpu.sync_copy(ones_hbm.at[pl.ds(0, CK)], ones_v)
        plsc.subcore_barrier()
        wid = s * NC + c

        def superstep(cb, nb):
            pltpu.sync_copy(col2_hbm.at[pl.ds(cb, nb)], cbuf.at[pl.ds(0, nb)])
            sd = [pltpu.async_copy(ones_v, acc.at[cbuf.at[j]], ssem, add=True)
                  for j in range(nb)]
            for d in sd:
                d.wait()

        cb0 = wid * CPT

        def body(i, carry):
            superstep(cb0 + i * SB, SB)
            return carry

        lax.fori_loop(0, nss, body, 0)
        if rem:
            superstep(cb0 + nss * SB, rem)
        if XT:
            @pl.when(wid < XT)
            def _():
                superstep(EC - XT + wid, 1)
        plsc.subcore_barrier()
        pltpu.sync_copy(acc.at[pl.ds(r0, RT)], out.at[c, pl.ds(r0, RT)])

    return kern


@functools.lru_cache(maxsize=None)
def _edge_pass(N, E, D):
    EC = E // CK
    CPT = EC // NS
    nss, rem = CPT // SB, CPT % SB
    XT = EC - CPT * NS
    RT = N // NS

    @functools.partial(
        pl.kernel,
        out_type=(jax.ShapeDtypeStruct((N, D), jnp.float32),
                  jax.ShapeDtypeStruct((N, D), jnp.float32)),
        mesh=_sc_mesh(),
        compiler_params=_SC_PARAMS,
        scratch_types=[
            pltpu.VMEM((SB, CK), jnp.int32),
            pltpu.VMEM((SB, CK), jnp.int32),
            pltpu.VMEM((SB, CK, D), jnp.float32),
            pltpu.VMEM_SHARED((N, D), jnp.float32),
            pltpu.SemaphoreType.DMA,
            pltpu.SemaphoreType.DMA,
        ],
    )
    def kern(g_lo, g_hi, row2_hbm, col2_hbm, out_lo, out_hi,
             rbuf, cbuf, msg, acc, gsem, ssem):
        c = lax.axis_index("c")
        s = lax.axis_index("s")

        def run(gref, oref):
            r0 = s * RT
            pltpu.sync_copy(gref.at[pl.ds(r0, RT)], acc.at[pl.ds(r0, RT)])
            plsc.subcore_barrier()

            def superstep(cb, nb):
                pltpu.sync_copy(row2_hbm.at[pl.ds(cb, nb)],
                                rbuf.at[pl.ds(0, nb)])
                pltpu.sync_copy(col2_hbm.at[pl.ds(cb, nb)],
                                cbuf.at[pl.ds(0, nb)])
                gd = [pltpu.async_copy(gref.at[rbuf.at[j]], msg.at[j], gsem)
                      for j in range(nb)]
                for d in gd:
                    d.wait()
                sd = [pltpu.async_copy(msg.at[j], acc.at[cbuf.at[j]], ssem,
                                       add=True)
                      for j in range(nb)]
                for d in sd:
                    d.wait()

            cb0 = s * CPT

            def body(i, carry):
                superstep(cb0 + i * SB, SB)
                return carry

            lax.fori_loop(0, nss, body, 0)
            if rem:
                superstep(cb0 + nss * SB, rem)
            if XT:
                @pl.when(s < XT)
                def _():
                    superstep(EC - XT + s, 1)
            plsc.subcore_barrier()
            pltpu.sync_copy(acc.at[pl.ds(r0, RT)], oref.at[pl.ds(r0, RT)])

        @pl.when(c == 0)
        def _():
            run(g_lo, out_lo)

        @pl.when(c == 1)
        def _():
            run(g_hi, out_hi)

    return kern


BLK = 5000


@functools.lru_cache(maxsize=None)
def _prep1(N, Np, F):
    H = F // 2
    grid = (N // BLK,)

    def body(x_ref, w_ref, p0_ref, p1_ref, lo_ref, hi_ref, dinv_ref):
        deg = p0_ref[...] + p1_ref[...] - 1.0
        dinv = lax.rsqrt(deg)
        h = lax.dot_general(x_ref[...], w_ref[...],
                            (((1,), (0,)), ((), ())),
                            preferred_element_type=jnp.float32)
        g = h * dinv
        lo_ref[...] = g[:, :H]
        hi_ref[...] = g[:, H:]
        dinv_ref[...] = dinv

    return pl.pallas_call(
        body,
        grid=grid,
        in_specs=[
            pl.BlockSpec((BLK, 8), lambda i: (i, 0)),
            pl.BlockSpec((8, F), lambda i: (0, 0)),
            pl.BlockSpec((BLK, 1), lambda i: (i, 0)),
            pl.BlockSpec((BLK, 1), lambda i: (i, 0)),
        ],
        out_specs=[
            pl.BlockSpec((BLK, H), lambda i: (i, 0)),
            pl.BlockSpec((BLK, H), lambda i: (i, 0)),
            pl.BlockSpec((BLK, 1), lambda i: (i, 0)),
        ],
        out_shape=[
            jax.ShapeDtypeStruct((Np, H), jnp.float32),
            jax.ShapeDtypeStruct((Np, H), jnp.float32),
            jax.ShapeDtypeStruct((N, 1), jnp.float32),
        ],
    )


@functools.lru_cache(maxsize=None)
def _mid(N, Np, F1, F2):
    H1, H2 = F1 // 2, F2 // 2
    grid = (N // BLK,)

    def body(lo_ref, hi_ref, dinv_ref, b1_ref, w2_ref, olo_ref, ohi_ref):
        accv = jnp.concatenate([lo_ref[...], hi_ref[...]], axis=1)
        dinv = dinv_ref[...]
        out1 = jnp.maximum(accv * dinv + b1_ref[...], 0.0)
        h2 = lax.dot_general(out1, w2_ref[...],
                             (((1,), (0,)), ((), ())),
                             preferred_element_type=jnp.float32)
        g2 = h2 * dinv
        olo_ref[...] = g2[:, :H2]
        ohi_ref[...] = g2[:, H2:]

    return pl.pallas_call(
        body,
        grid=grid,
        in_specs=[
            pl.BlockSpec((BLK, H1), lambda i: (i, 0)),
            pl.BlockSpec((BLK, H1), lambda i: (i, 0)),
            pl.BlockSpec((BLK, 1), lambda i: (i, 0)),
            pl.BlockSpec((1, F1), lambda i: (0, 0)),
            pl.BlockSpec((F1, F2), lambda i: (0, 0)),
        ],
        out_specs=[
            pl.BlockSpec((BLK, H2), lambda i: (i, 0)),
            pl.BlockSpec((BLK, H2), lambda i: (i, 0)),
        ],
        out_shape=[
            jax.ShapeDtypeStruct((Np, H2), jnp.float32),
            jax.ShapeDtypeStruct((Np, H2), jnp.float32),
        ],
    )


@functools.lru_cache(maxsize=None)
def _tail(N, Np, F2, G):
    H2 = F2 // 2
    grid = (N // BLK,)
    ng = N // BLK

    def body(lo_ref, hi_ref, dinv_ref, b2_ref, bat_ref, w3_ref, b3_ref,
             out_ref, acc_ref):
        pid = pl.program_id(0)

        @pl.when(pid == 0)
        def _():
            acc_ref[...] = jnp.zeros_like(acc_ref)

        accv = jnp.concatenate([lo_ref[...], hi_ref[...]], axis=1)
        h = jnp.maximum(accv * dinv_ref[...] + b2_ref[...], 0.0)
        h1 = jnp.concatenate([h, jnp.ones((BLK, 1), jnp.float32)], axis=1)
        seg = lax.broadcasted_iota(jnp.int32, (1, G), 1)
        onehot = (bat_ref[...] == seg).astype(jnp.float32)
        acc_ref[...] += lax.dot_general(onehot, h1,
                                        (((0,), (0,)), ((), ())),
                                        preferred_element_type=jnp.float32)

        @pl.when(pid == ng - 1)
        def _():
            a = acc_ref[...]
            sums = a[:, :F2]
            cnt = a[:, F2:F2 + 1]
            pooled = sums / jnp.maximum(cnt, 1.0)
            z = lax.dot_general(pooled, w3_ref[...],
                                (((1,), (0,)), ((), ())),
                                preferred_element_type=jnp.float32) + b3_ref[...]
            out_ref[...] = 1.0 / (1.0 + jnp.exp(-z))

    return pl.pallas_call(
        body,
        grid=grid,
        in_specs=[
            pl.BlockSpec((BLK, H2), lambda i: (i, 0)),
            pl.BlockSpec((BLK, H2), lambda i: (i, 0)),
            pl.BlockSpec((BLK, 1), lambda i: (i, 0)),
            pl.BlockSpec((1, F2), lambda i: (0, 0)),
            pl.BlockSpec((BLK, 1), lambda i: (i, 0)),
            pl.BlockSpec((F2, 1), lambda i: (0, 0)),
            pl.BlockSpec((1, 1), lambda i: (0, 0)),
        ],
        out_specs=pl.BlockSpec((G, 1), lambda i: (0, 0)),
        out_shape=jax.ShapeDtypeStruct((G, 1), jnp.float32),
        scratch_shapes=[pltpu.VMEM((G, F2 + 1), jnp.float32)],
    )


def kernel(x, edge_index, batch, W1, b1, W2, b2, W3, b3):
    N = x.shape[0]
    E = edge_index.shape[1]
    G = 64
    row2 = edge_index[0].reshape(E // CK, CK)
    col2 = edge_index[1].reshape(E // CK, CK)
    Np = ((N + 127) // 128) * 128
    ones = jnp.ones((Np, 1), jnp.float32)

    degp = _deg_pass(Np, E)(col2, ones)
    g1lo, g1hi, dinv = _prep1(N, Np, 16)(x, W1, degp[0], degp[1])
    s1lo, s1hi = _edge_pass(Np, E, 8)(g1lo, g1hi, row2, col2)
    g2lo, g2hi = _mid(N, Np, 16, 24)(s1lo, s1hi, dinv, b1.reshape(1, 16), W2)
    s2lo, s2hi = _edge_pass(Np, E, 12)(g2lo, g2hi, row2, col2)
    out = _tail(N, Np, 24, G)(s2lo, s2hi, dinv, b2.reshape(1, 24),
                              batch.reshape(N, 1), W3, b3.reshape(1, 1))
    return out
